# baseline (speedup 1.0000x reference)
import jax
import jax.numpy as jnp
from jax.experimental import pallas as pl

N_EMBD = 4096
HIDDEN = N_EMBD // 4
N_EXPERTS = 64
TOP_K = 8
BM = 1024
RCHUNK = 4


def _router_block(x_ref, w1_ref, w2_ref, ew_ref, mask_ref, probs_ref):
    x = x_ref[...]
    h = jax.lax.dot_general(
        x, w1_ref[...], (((1,), (1,)), ((), ())),
        preferred_element_type=jnp.float32)
    h = h * jax.nn.sigmoid(h)
    logits = jax.lax.dot_general(
        w2_ref[...], h, (((1,), (1,)), ((), ())),
        preferred_element_type=jnp.float32)

    cw = BM // RCHUNK
    inv_idx = jnp.int32(N_EXPERTS - 1) - jax.lax.broadcasted_iota(
        jnp.int32, (N_EXPERTS, cw), 0)
    for c in range(RCHUNK):
        cols = slice(c * cw, (c + 1) * cw)
        lg = logits[:, cols]

        mx = jnp.max(lg, axis=0, keepdims=True)
        e = jnp.exp(lg - mx)
        probs = e / jnp.sum(e, axis=0, keepdims=True)
        probs_ref[:, cols] = probs

        work = probs
        mask = jnp.zeros_like(probs)
        for _ in range(TOP_K):
            m = jnp.max(work, axis=0, keepdims=True)
            is_max = work == m
            cand = jnp.where(is_max, inv_idx, -1)
            win = jnp.max(cand, axis=0, keepdims=True)
            sel = cand == win
            mask = mask + sel.astype(jnp.float32)
            work = jnp.where(sel, -1.0, work)
        mask_ref[:, cols] = mask

        masked = probs * mask
        wsum = jnp.maximum(jnp.sum(masked, axis=0, keepdims=True), 1e-8)
        ew_ref[:, cols] = masked / wsum


def kernel(x, W1, W2):
    B, T, E = x.shape
    M = B * T
    xf = x.reshape(M, E)
    outs = pl.pallas_call(
        _router_block,
        grid=(M // BM,),
        in_specs=[
            pl.BlockSpec((BM, E), lambda i: (i, 0)),
            pl.BlockSpec((HIDDEN, E), lambda i: (0, 0)),
            pl.BlockSpec((N_EXPERTS, HIDDEN), lambda i: (0, 0)),
        ],
        out_specs=[pl.BlockSpec((N_EXPERTS, BM), lambda i: (0, i))] * 3,
        out_shape=[jax.ShapeDtypeStruct((N_EXPERTS, M), jnp.float32)] * 3,
    )(xf, W1, W2)
    ew, mask, probs = (o.T.reshape(B, T, N_EXPERTS) for o in outs)
    return (ew, mask, probs)
